# Initial kernel scaffold; baseline (speedup 1.0000x reference)
import jax
import jax.numpy as jnp
from jax.experimental import pallas as pl


def kernel(mov, ref, defgrad):
    raise NotImplementedError("write your pallas kernel here")


# speedup vs baseline: 1.2200x; 1.2200x over previous
import functools

import jax
import jax.numpy as jnp
from jax import lax
from jax.experimental import pallas as pl
from jax.experimental.pallas import tpu as pltpu
from jax.experimental.pallas import tpu_sc as plsc

B, H, W, C = 2, 384, 384, 96
N = B * H * W
HW = H * W
K = 128
NCH = N // K
NW = 32
CPW = NCH // NW
NLANE = 16


_SC_COLS = 4608


def _grid_body(din_ref, cs_ref, norm_ref, inv_ref):
    dg = 2.0 / (1.0 + jnp.exp(-din_ref[...]))
    prev = dg[0:1, :]
    cs_ref[0:1, :] = prev
    for j in range(1, 128):
        prev = prev + dg[j:j + 1, :]
        cs_ref[j:j + 1, :] = prev
    lane = lax.broadcasted_iota(jnp.int32, (1, _SC_COLS), 1)
    m1 = lax.rem(lane, 3)
    z1 = jnp.zeros((1, 1), jnp.float32)
    z2 = jnp.zeros((1, 2), jnp.float32)

    sums = cs_ref[127:128, :]
    sr1 = jnp.concatenate([z1, sums[:, :-1]], axis=1)
    sr2 = jnp.concatenate([z2, sums[:, :-2]], axis=1)
    off = jnp.where(m1 == 0, 0.0, jnp.where(m1 == 1, sr1, sr2 + sr1))
    cs = cs_ref[...] + off

    g = cs - 1.0
    v0 = g[0:1, :]
    v1 = g[127:128, :]
    v0r1 = jnp.concatenate([z1, v0[:, :-1]], axis=1)
    v0r2 = jnp.concatenate([z2, v0[:, :-2]], axis=1)
    lo = jnp.where(m1 == 0, v0, jnp.where(m1 == 1, v0r1, v0r2))
    v1l1 = jnp.concatenate([v1[:, 1:], z1], axis=1)
    v1l2 = jnp.concatenate([v1[:, 2:], z2], axis=1)
    hi = jnp.where(m1 == 0, v1l2, jnp.where(m1 == 1, v1l1, v1))

    norm = jnp.float32(W - 1) * ((g - lo) / (hi - lo))
    mfull = lax.rem(lax.broadcasted_iota(jnp.int32, (128, _SC_COLS), 1), 3)
    jfull = lax.broadcasted_iota(jnp.int32, (128, _SC_COLS), 0)
    basef = (mfull * 128 + jfull).astype(jnp.float32)
    norm_ref[...] = norm
    inv_ref[...] = 2.0 * basef - norm


_RB = 256


def _coef_body(x_ref, y_ref, idx_ref, wt_ref):
    x = x_ref[...]
    y = y_ref[...]
    r0 = pl.program_id(0) * _RB
    row = lax.broadcasted_iota(jnp.int32, (_RB, K), 0) + r0
    rr = lax.rem(row, jnp.int32(NCH))
    base = jnp.where(rr >= NCH // B, jnp.int32(HW), jnp.int32(0))

    x0r = jnp.floor(x).astype(jnp.int32)
    y0r = jnp.floor(y).astype(jnp.int32)
    x0 = jnp.clip(x0r, 0, W - 1)
    x1 = jnp.clip(x0r + 1, 0, W - 1)
    y0 = jnp.clip(y0r, 0, H - 1)
    y1 = jnp.clip(y0r + 1, 0, H - 1)
    idx_ref[:, 0, :] = base + y0 * W + x0
    idx_ref[:, 1, :] = base + y1 * W + x0
    idx_ref[:, 2, :] = base + y0 * W + x1
    idx_ref[:, 3, :] = base + y1 * W + x1

    x0f = x0.astype(jnp.float32)
    x1f = x1.astype(jnp.float32)
    y0f = y0.astype(jnp.float32)
    y1f = y1.astype(jnp.float32)
    wt_ref[:, 0, :] = (x1f - x) * (y1f - y)
    wt_ref[:, 1, :] = (x1f - x) * (y - y0f)
    wt_ref[:, 2, :] = (x - x0f) * (y1f - y)
    wt_ref[:, 3, :] = (x - x0f) * (y - y0f)


def _resample_body(tab, idx_hbm, wt_hbm, out, idxb, wb, rows, outb, sem):
    wid = lax.axis_index("s") * 2 + lax.axis_index("c")
    c0 = wid * CPW

    def chunk_body(g, _):
        c = c0 + g
        pltpu.sync_copy(idx_hbm.at[c], idxb)
        pltpu.sync_copy(wt_hbm.at[c], wb)
        cps = [pltpu.async_copy(tab.at[idxb.at[j]], rows.at[j], sem)
               for j in range(4)]
        for cp in cps:
            cp.wait()

        def point_body(i, _):
            i16 = jnp.full((NLANE,), i, jnp.int32)
            ws = [plsc.load_gather(wb, [i16 + (j * K)]) for j in range(4)]
            for cc in range(C // NLANE):
                sl = pl.ds(cc * NLANE, NLANE)
                acc = ws[0] * rows[0, i, sl]
                acc += ws[1] * rows[1, i, sl]
                acc += ws[2] * rows[2, i, sl]
                acc += ws[3] * rows[3, i, sl]
                outb[i, sl] = acc
            return 0

        lax.fori_loop(0, K, point_body, 0)
        pltpu.sync_copy(outb, out.at[pl.ds(c * K, K)])
        return 0

    lax.fori_loop(0, CPW, chunk_body, 0)


@functools.lru_cache(maxsize=None)
def _make_resample():
    return pl.kernel(
        _resample_body,
        out_type=jax.ShapeDtypeStruct((N, C), jnp.float32),
        mesh=plsc.VectorSubcoreMesh(core_axis_name="c", subcore_axis_name="s"),
        compiler_params=pltpu.CompilerParams(needs_layout_passes=False,
                                             use_tc_tiling_on_sc=False),
        scratch_types=[
            pltpu.VMEM((4, K), jnp.int32),
            pltpu.VMEM((4 * K,), jnp.float32),
            pltpu.VMEM((4, K, C), jnp.float32),
            pltpu.VMEM((K, C), jnp.float32),
            pltpu.SemaphoreType.DMA,
        ],
    )


def kernel(mov, ref, defgrad):
    del ref
    xin = defgrad[..., 0].reshape(B * H, 3, 128).transpose(2, 0, 1)
    yin = (defgrad[..., 1].transpose(0, 2, 1)
           .reshape(B * W, 3, 128).transpose(2, 0, 1))
    scan_in = jnp.concatenate([xin.reshape(128, 3 * B * H),
                               yin.reshape(128, 3 * B * W)], axis=1)

    _, norm_s, inv_s = pl.pallas_call(
        _grid_body,
        out_shape=[
            jax.ShapeDtypeStruct((128, _SC_COLS), jnp.float32),
            jax.ShapeDtypeStruct((128, _SC_COLS), jnp.float32),
            jax.ShapeDtypeStruct((128, _SC_COLS), jnp.float32),
        ],
    )(scan_in)

    def unscan(arr_half):
        return arr_half.reshape(128, 768, 3).transpose(1, 2, 0).reshape(768, 384)

    nx = unscan(norm_s[:, :2304])
    ix = unscan(inv_s[:, :2304])
    nyt = unscan(norm_s[:, 2304:])
    iyt = unscan(inv_s[:, 2304:])
    ny = nyt.reshape(B, W, H).transpose(0, 2, 1)
    iy = iyt.reshape(B, W, H).transpose(0, 2, 1)
    nxr = nx.reshape(B, H, W)
    ixr = ix.reshape(B, H, W)
    norm = jnp.stack([nxr, ny], axis=-1)
    inv = jnp.stack([ixr, iy], axis=-1)

    X = jnp.concatenate([nx.reshape(-1), ix.reshape(-1)]).reshape(2 * NCH, K)
    Y = jnp.concatenate([ny.reshape(-1), iy.reshape(-1)]).reshape(2 * NCH, K)

    idx, wt = pl.pallas_call(
        _coef_body,
        grid=(2 * NCH // _RB,),
        in_specs=[
            pl.BlockSpec((_RB, K), lambda i: (i, 0)),
            pl.BlockSpec((_RB, K), lambda i: (i, 0)),
        ],
        out_specs=[
            pl.BlockSpec((_RB, 4, K), lambda i: (i, 0, 0)),
            pl.BlockSpec((_RB, 4, K), lambda i: (i, 0, 0)),
        ],
        out_shape=[
            jax.ShapeDtypeStruct((2 * NCH, 4, K), jnp.int32),
            jax.ShapeDtypeStruct((2 * NCH, 4, K), jnp.float32),
        ],
    )(X, Y)

    resample = _make_resample()
    mov_flat = mov.reshape(N, C)
    wtf = wt.reshape(2 * NCH, 4 * K)
    movdef_flat = resample(mov_flat, idx[:NCH], wtf[:NCH])
    refdef_flat = resample(movdef_flat, idx[NCH:], wtf[NCH:])

    return (movdef_flat.reshape(B, H, W, C),
            refdef_flat.reshape(B, H, W, C),
            norm, inv)


# speedup vs baseline: 1.6094x; 1.3191x over previous
import functools

import jax
import jax.numpy as jnp
from jax import lax
from jax.experimental import pallas as pl
from jax.experimental.pallas import tpu as pltpu
from jax.experimental.pallas import tpu_sc as plsc

B, H, W, C = 2, 384, 384, 96
N = B * H * W
HW = H * W
K = 96
NCH = N // K
NW = 32
CPW = NCH // NW
NLANE = 16


_SC_COLS = 4608


def _grid_body(din_ref, cs_ref, norm_ref, inv_ref):
    dg = 2.0 / (1.0 + jnp.exp(-din_ref[...]))
    prev = dg[0:1, :]
    cs_ref[0:1, :] = prev
    for j in range(1, 128):
        prev = prev + dg[j:j + 1, :]
        cs_ref[j:j + 1, :] = prev
    lane = lax.broadcasted_iota(jnp.int32, (1, _SC_COLS), 1)
    m1 = lax.rem(lane, 3)
    z1 = jnp.zeros((1, 1), jnp.float32)
    z2 = jnp.zeros((1, 2), jnp.float32)

    sums = cs_ref[127:128, :]
    sr1 = jnp.concatenate([z1, sums[:, :-1]], axis=1)
    sr2 = jnp.concatenate([z2, sums[:, :-2]], axis=1)
    off = jnp.where(m1 == 0, 0.0, jnp.where(m1 == 1, sr1, sr2 + sr1))
    cs = cs_ref[...] + off

    g = cs - 1.0
    v0 = g[0:1, :]
    v1 = g[127:128, :]
    v0r1 = jnp.concatenate([z1, v0[:, :-1]], axis=1)
    v0r2 = jnp.concatenate([z2, v0[:, :-2]], axis=1)
    lo = jnp.where(m1 == 0, v0, jnp.where(m1 == 1, v0r1, v0r2))
    v1l1 = jnp.concatenate([v1[:, 1:], z1], axis=1)
    v1l2 = jnp.concatenate([v1[:, 2:], z2], axis=1)
    hi = jnp.where(m1 == 0, v1l2, jnp.where(m1 == 1, v1l1, v1))

    norm = jnp.float32(W - 1) * ((g - lo) / (hi - lo))
    mfull = lax.rem(lax.broadcasted_iota(jnp.int32, (128, _SC_COLS), 1), 3)
    jfull = lax.broadcasted_iota(jnp.int32, (128, _SC_COLS), 0)
    basef = (mfull * 128 + jfull).astype(jnp.float32)
    norm_ref[...] = norm
    inv_ref[...] = 2.0 * basef - norm


_RB = 256


def _coef_body(x_ref, y_ref, idx_ref, wt_ref):
    x = x_ref[...]
    y = y_ref[...]
    r0 = pl.program_id(0) * _RB
    row = lax.broadcasted_iota(jnp.int32, (_RB, K), 0) + r0
    rr = lax.rem(row, jnp.int32(NCH))
    base = jnp.where(rr >= NCH // B, jnp.int32(HW), jnp.int32(0))

    x0r = jnp.floor(x).astype(jnp.int32)
    y0r = jnp.floor(y).astype(jnp.int32)
    x0 = jnp.clip(x0r, 0, W - 1)
    x1 = jnp.clip(x0r + 1, 0, W - 1)
    y0 = jnp.clip(y0r, 0, H - 1)
    y1 = jnp.clip(y0r + 1, 0, H - 1)
    idx_ref[:, 0, :] = base + y0 * W + x0
    idx_ref[:, 1, :] = base + y1 * W + x0
    idx_ref[:, 2, :] = base + y0 * W + x1
    idx_ref[:, 3, :] = base + y1 * W + x1

    x0f = x0.astype(jnp.float32)
    x1f = x1.astype(jnp.float32)
    y0f = y0.astype(jnp.float32)
    y1f = y1.astype(jnp.float32)
    wt_ref[:, 0, :] = (x1f - x) * (y1f - y)
    wt_ref[:, 1, :] = (x1f - x) * (y - y0f)
    wt_ref[:, 2, :] = (x - x0f) * (y1f - y)
    wt_ref[:, 3, :] = (x - x0f) * (y - y0f)


def _resample_body(tab, idx_hbm, wt_hbm, out,
                   idxb0, idxb1, wb0, wb1, rows0, rows1, outb0, outb1,
                   isem0, isem1, gsem0, gsem1, osem0, osem1):
    wid = lax.axis_index("s") * 2 + lax.axis_index("c")
    c0 = wid * CPW
    idxb = (idxb0, idxb1)
    wb = (wb0, wb1)
    rows = (rows0, rows1)
    outb = (outb0, outb1)
    isem = (isem0, isem1)
    gsem = (gsem0, gsem1)
    osem = (osem0, osem1)

    def fire_idx(g, s):
        pltpu.async_copy(idx_hbm.at[c0 + g], idxb[s], isem[s])
        pltpu.async_copy(wt_hbm.at[c0 + g], wb[s], isem[s])

    def wait_idx(s):
        pltpu.make_async_copy(idx_hbm.at[c0], idxb[s], isem[s]).wait()
        pltpu.make_async_copy(wt_hbm.at[c0], wb[s], isem[s]).wait()

    def fire_gathers(s):
        for j in range(4):
            pltpu.async_copy(tab.at[idxb[s].at[j]], rows[s].at[j], gsem[s])

    def wait_gathers(s):
        for j in range(4):
            pltpu.make_async_copy(tab.at[idxb[s].at[j]], rows[s].at[j],
                                  gsem[s]).wait()

    def fire_out(g, s):
        pltpu.async_copy(outb[s], out.at[pl.ds((c0 + g) * K, K)], osem[s])

    def wait_out(s):
        pltpu.make_async_copy(outb[s], out.at[pl.ds(c0 * K, K)],
                              osem[s]).wait()

    def compute(s):
        rs, ws, ob = rows[s], wb[s], outb[s]

        def point_body(i, _):
            i16 = jnp.full((NLANE,), i, jnp.int32)
            w4 = [plsc.load_gather(ws, [i16 + (j * K)]) for j in range(4)]
            for cc in range(C // NLANE):
                sl = pl.ds(cc * NLANE, NLANE)
                acc = w4[0] * rs[0, i, sl]
                acc += w4[1] * rs[1, i, sl]
                acc += w4[2] * rs[2, i, sl]
                acc += w4[3] * rs[3, i, sl]
                ob[i, sl] = acc
            return 0

        lax.fori_loop(0, K, point_body, 0)

    fire_idx(0, 0)
    fire_idx(1, 1)
    wait_idx(0)
    fire_gathers(0)

    def pair_body(p, _):
        for s in (0, 1):
            g = 2 * p + s
            o = 1 - s
            wait_gathers(s)

            @pl.when(g + 1 < CPW)
            def _():
                wait_idx(o)
                fire_gathers(o)

            @pl.when(g >= 2)
            def _():
                wait_out(s)

            compute(s)
            fire_out(g, s)

            @pl.when(g + 2 < CPW)
            def _():
                fire_idx(g + 2, s)

        return 0

    lax.fori_loop(0, CPW // 2, pair_body, 0)
    wait_out(0)
    wait_out(1)


@functools.lru_cache(maxsize=None)
def _make_resample():
    return pl.kernel(
        _resample_body,
        out_type=jax.ShapeDtypeStruct((N, C), jnp.float32),
        mesh=plsc.VectorSubcoreMesh(core_axis_name="c", subcore_axis_name="s"),
        compiler_params=pltpu.CompilerParams(needs_layout_passes=False,
                                             use_tc_tiling_on_sc=False),
        scratch_types=[
            pltpu.VMEM((4, K), jnp.int32),
            pltpu.VMEM((4, K), jnp.int32),
            pltpu.VMEM((4 * K,), jnp.float32),
            pltpu.VMEM((4 * K,), jnp.float32),
            pltpu.VMEM((4, K, C), jnp.float32),
            pltpu.VMEM((4, K, C), jnp.float32),
            pltpu.VMEM((K, C), jnp.float32),
            pltpu.VMEM((K, C), jnp.float32),
            pltpu.SemaphoreType.DMA,
            pltpu.SemaphoreType.DMA,
            pltpu.SemaphoreType.DMA,
            pltpu.SemaphoreType.DMA,
            pltpu.SemaphoreType.DMA,
            pltpu.SemaphoreType.DMA,
        ],
    )


def kernel(mov, ref, defgrad):
    del ref
    xin = defgrad[..., 0].reshape(B * H, 3, 128).transpose(2, 0, 1)
    yin = (defgrad[..., 1].transpose(0, 2, 1)
           .reshape(B * W, 3, 128).transpose(2, 0, 1))
    scan_in = jnp.concatenate([xin.reshape(128, 3 * B * H),
                               yin.reshape(128, 3 * B * W)], axis=1)

    _, norm_s, inv_s = pl.pallas_call(
        _grid_body,
        out_shape=[
            jax.ShapeDtypeStruct((128, _SC_COLS), jnp.float32),
            jax.ShapeDtypeStruct((128, _SC_COLS), jnp.float32),
            jax.ShapeDtypeStruct((128, _SC_COLS), jnp.float32),
        ],
    )(scan_in)

    def unscan(arr_half):
        return arr_half.reshape(128, 768, 3).transpose(1, 2, 0).reshape(768, 384)

    nx = unscan(norm_s[:, :2304])
    ix = unscan(inv_s[:, :2304])
    nyt = unscan(norm_s[:, 2304:])
    iyt = unscan(inv_s[:, 2304:])
    ny = nyt.reshape(B, W, H).transpose(0, 2, 1)
    iy = iyt.reshape(B, W, H).transpose(0, 2, 1)
    nxr = nx.reshape(B, H, W)
    ixr = ix.reshape(B, H, W)
    norm = jnp.stack([nxr, ny], axis=-1)
    inv = jnp.stack([ixr, iy], axis=-1)

    X = jnp.concatenate([nx.reshape(-1), ix.reshape(-1)]).reshape(2 * NCH, K)
    Y = jnp.concatenate([ny.reshape(-1), iy.reshape(-1)]).reshape(2 * NCH, K)

    idx, wt = pl.pallas_call(
        _coef_body,
        grid=(2 * NCH // _RB,),
        in_specs=[
            pl.BlockSpec((_RB, K), lambda i: (i, 0)),
            pl.BlockSpec((_RB, K), lambda i: (i, 0)),
        ],
        out_specs=[
            pl.BlockSpec((_RB, 4, K), lambda i: (i, 0, 0)),
            pl.BlockSpec((_RB, 4, K), lambda i: (i, 0, 0)),
        ],
        out_shape=[
            jax.ShapeDtypeStruct((2 * NCH, 4, K), jnp.int32),
            jax.ShapeDtypeStruct((2 * NCH, 4, K), jnp.float32),
        ],
    )(X, Y)

    resample = _make_resample()
    mov_flat = mov.reshape(N, C)
    wtf = wt.reshape(2 * NCH, 4 * K)
    movdef_flat = resample(mov_flat, idx[:NCH], wtf[:NCH])
    refdef_flat = resample(movdef_flat, idx[NCH:], wtf[NCH:])

    return (movdef_flat.reshape(B, H, W, C),
            refdef_flat.reshape(B, H, W, C),
            norm, inv)
